# baseline (speedup 1.0000x reference)
import functools

import jax
import jax.numpy as jnp
from jax import lax
from jax.experimental import pallas as pl
from jax.experimental.pallas import tpu as pltpu
from jax.experimental.pallas import tpu_sc as plsc

N = 10000
E = 320000
D = 128
L = 16
NC = 2
NS = 16
NP = NS * 640
CPT = D // NS
PPT = CPT // 2
EPC = E // NC
CE = 640
NCH = EPC // CE
GRP = CE // L
DSUB = CE // 80


def _sc_body(xp_hbm, edata_hbm, dstd_hbm, valsf_hbm, p_hbm, deg_hbm,
             xpair_v, acc_v, ebuf0, ebuf1, ddst_v, vf_v, zdeg_v,
             dacc_sh, esem0, esem1, dsem):
    cid = lax.axis_index("c")
    sid = lax.axis_index("s")

    pltpu.sync_copy(xp_hbm.at[sid], xpair_v)

    zeros16 = jnp.zeros((L,), jnp.float32)

    def zacc(i, carry):
        for r in range(CPT):
            acc_v[r, pl.ds(i * L, L)] = zeros16
        return carry

    lax.fori_loop(0, NP // L, zacc, 0)

    for j in range(640 // L):
        zdeg_v[pl.ds(j * L, L)] = zeros16
    doff = pl.multiple_of(sid * 640, 8)
    pltpu.sync_copy(zdeg_v, dacc_sh.at[pl.ds(doff, 640)])
    plsc.subcore_barrier()

    idxc = [jnp.full((L,), c, jnp.int32) for c in range(CPT)]

    def process(k, ebuf):
        def group_body(g, carry):
            base = g * L
            src16 = ebuf[0, pl.ds(base, L)]
            dst16 = ebuf[1, pl.ds(base, L)]
            val16 = plsc.bitcast(ebuf[2, pl.ds(base, L)], jnp.float32)
            for pc in range(PPT):
                pair = plsc.load_gather(xpair_v, [idxc[pc], src16])
                two = plsc.bitcast(pair, jnp.bfloat16)
                a, b = plsc.unpack(two, format=plsc.PackFormat.INTERLEAVED)
                plsc.addupdate_scatter(acc_v, [idxc[2 * pc], dst16],
                                       a * val16)
                plsc.addupdate_scatter(acc_v, [idxc[2 * pc + 1], dst16],
                                       b * val16)
            return carry

        lax.fori_loop(0, GRP, group_body, 0)

        @pl.when(sid == lax.rem(k, NS))
        def _():
            @pl.when(k >= NS)
            def _():
                for j in range(DSUB):
                    pltpu.make_async_copy(
                        vf_v.at[j], dacc_sh.at[ddst_v.at[j]], dsem).wait()

            pltpu.sync_copy(dstd_hbm.at[cid, k], ddst_v)
            pltpu.sync_copy(valsf_hbm.at[cid, k], vf_v)
            for j in range(DSUB):
                pltpu.async_copy(vf_v.at[j], dacc_sh.at[ddst_v.at[j]],
                                 dsem, add=True)

    pltpu.async_copy(edata_hbm.at[cid, 0], ebuf0, esem0)

    def chunk_iter(k, carry):
        @pl.when(k % 2 == 0)
        def _():
            @pl.when(k < NCH - 1)
            def _():
                pltpu.async_copy(edata_hbm.at[cid, k + 1], ebuf1, esem1)
            pltpu.make_async_copy(edata_hbm.at[cid, k], ebuf0, esem0).wait()
            process(k, ebuf0)

        @pl.when(k % 2 == 1)
        def _():
            @pl.when(k < NCH - 1)
            def _():
                pltpu.async_copy(edata_hbm.at[cid, k + 1], ebuf0, esem0)
            pltpu.make_async_copy(edata_hbm.at[cid, k], ebuf1, esem1).wait()
            process(k, ebuf1)

        return carry

    lax.fori_loop(0, NCH, chunk_iter, 0)

    for j in range(DSUB):
        pltpu.make_async_copy(
            vf_v.at[j], dacc_sh.at[ddst_v.at[j]], dsem).wait()
    plsc.subcore_barrier()

    coff = pl.multiple_of(sid * CPT, 8)
    pltpu.sync_copy(acc_v, p_hbm.at[cid, pl.ds(coff, CPT)])
    pltpu.sync_copy(dacc_sh.at[pl.ds(doff, 640)],
                    deg_hbm.at[cid, pl.ds(doff, 640)])


_sc_scatter = functools.partial(
    pl.kernel,
    out_type=[
        jax.ShapeDtypeStruct((NC, D, NP), jnp.float32),
        jax.ShapeDtypeStruct((NC, NP), jnp.float32),
    ],
    mesh=plsc.VectorSubcoreMesh(core_axis_name="c", subcore_axis_name="s"),
    compiler_params=pltpu.CompilerParams(needs_layout_passes=False),
    scratch_types=[
        pltpu.VMEM((PPT, N), jnp.int32),
        pltpu.VMEM((CPT, NP), jnp.float32),
        pltpu.VMEM((3, CE), jnp.int32),
        pltpu.VMEM((3, CE), jnp.int32),
        pltpu.VMEM((DSUB, 80), jnp.int32),
        pltpu.VMEM((DSUB, 80), jnp.float32),
        pltpu.VMEM((640,), jnp.float32),
        pltpu.VMEM_SHARED((NP,), jnp.float32),
        pltpu.SemaphoreType.DMA,
        pltpu.SemaphoreType.DMA,
        pltpu.SemaphoreType.DMA,
    ],
)(_sc_body)


def _mm_body(p0_ref, p1_ref, d0_ref, d1_ref, w_ref, b_ref, o_ref):
    ht = p0_ref[...] + p1_ref[...]
    dd = d0_ref[...] + d1_ref[...]
    o_ref[...] = (lax.dot_general(ht, w_ref[...], (((0,), (1,)), ((), ())),
                                  preferred_element_type=jnp.float32)
                  + dd * b_ref[...])


_R = 2048


def _tc_matmul(p0, p1, d0, d1, w, b2):
    return pl.pallas_call(
        _mm_body,
        grid=(NP // _R,),
        in_specs=[
            pl.BlockSpec((D, _R), lambda i: (0, i)),
            pl.BlockSpec((D, _R), lambda i: (0, i)),
            pl.BlockSpec((_R, 1), lambda i: (i, 0)),
            pl.BlockSpec((_R, 1), lambda i: (i, 0)),
            pl.BlockSpec((D, D), lambda i: (0, 0)),
            pl.BlockSpec((1, D), lambda i: (0, 0)),
        ],
        out_specs=pl.BlockSpec((_R, D), lambda i: (i, 0)),
        out_shape=jax.ShapeDtypeStruct((NP, D), jnp.float32),
    )(p0, p1, d0, d1, w, b2)


def kernel(x, edge_index, adj_vals, W, b):
    ei = edge_index.astype(jnp.int32)
    xb = jax.lax.bitcast_convert_type(
        x.astype(jnp.bfloat16).reshape(N, D // 2, 2), jnp.int32)
    xp = jnp.transpose(xb, (1, 0)).reshape(NS, PPT, N)
    vals_i = jax.lax.bitcast_convert_type(adj_vals, jnp.int32)
    edata = jnp.stack([
        ei[1].reshape(NC, NCH, CE),
        ei[0].reshape(NC, NCH, CE),
        vals_i.reshape(NC, NCH, CE),
    ], axis=2)
    dstd = ei[0].reshape(NC, NCH, DSUB, 80)
    valsf = adj_vals.reshape(NC, NCH, DSUB, 80)
    P, deg = _sc_scatter(xp, edata, dstd, valsf)
    out = _tc_matmul(P[0], P[1], deg[0][:, None], deg[1][:, None],
                     W, b[None, :])
    return out[:N]
